# Initial kernel scaffold; baseline (speedup 1.0000x reference)
import jax
import jax.numpy as jnp
from jax.experimental import pallas as pl


def kernel(n_features, c_features, tables):
    raise NotImplementedError("write your pallas kernel here")


# speedup vs baseline: 2.4205x; 2.4205x over previous
import functools

import jax
import jax.numpy as jnp
from jax import lax
from jax.experimental import pallas as pl
from jax.experimental.pallas import tpu as pltpu
from jax.experimental.pallas import tpu_sc as plsc

N_FIELDS = 26
VOCAB = 100000
EMB_DIM = 16
B = 4096
L = 20
N_DENSE = 13

P = B * L
OUT_ROW = N_FIELDS * EMB_DIM + N_DENSE
NC, NS = 2, 16
NW = NC * NS
PW = P // NW
CHUNK = 128
NCHUNK = PW // CHUNK
IDXC = CHUNK * N_FIELDS
NVEC = IDXC // 16
ASM_WORDS = CHUNK * OUT_ROW
NF_WORDS = CHUNK * N_DENSE


def _sc_body(table_hbm, c_hbm, nf_hbm, out_hbm,
             raw_v, idx2d, gath, nf_v, asm, sem):
    wid = lax.axis_index("s") * NC + lax.axis_index("c")
    pos0 = wid * PW
    iota = lax.iota(jnp.int32, 16)
    mask13 = iota < N_DENSE

    def chunk_body(ci, carry):
        base = pos0 + ci * CHUNK

        pltpu.sync_copy(c_hbm.at[pl.ds(base * N_FIELDS, IDXC)], raw_v)
        pltpu.sync_copy(nf_hbm.at[pl.ds(base * N_DENSE, NF_WORDS)],
                        nf_v.at[pl.ds(0, NF_WORDS)])

        def idx_body(vi, c2):
            n0 = vi * 16
            n = iota + n0
            off = (n % N_FIELDS) * VOCAB
            v = raw_v[pl.ds(n0, 16)] + off
            idx2d[vi // 8, pl.ds((vi % 8) * 16, 16)] = v
            return c2
        lax.fori_loop(0, NVEC, idx_body, 0)

        copies = []
        for j in range(N_FIELDS):
            copies.append(pltpu.async_copy(
                table_hbm.at[idx2d.at[j]],
                gath.at[pl.ds(j * CHUNK, CHUNK)], sem))
        for cp in copies:
            cp.wait()

        def pos_body(p, c2):
            pr = p * OUT_ROW
            r0 = p * N_FIELDS
            for f in range(N_FIELDS):
                row = gath[r0 + f]
                plsc.store_scatter(asm, [pr + f * EMB_DIM + iota], row)
            dn = plsc.load_gather(nf_v, [p * N_DENSE + iota])
            plsc.store_scatter(asm, [pr + N_FIELDS * EMB_DIM + iota],
                               dn, mask=mask13)
            return c2
        lax.fori_loop(0, CHUNK, pos_body, 0)

        pltpu.sync_copy(asm, out_hbm.at[pl.ds(base * OUT_ROW, ASM_WORDS)])
        return carry

    lax.fori_loop(0, NCHUNK, chunk_body, 0)


@jax.jit
def _concat_emb(table_flat, c_flat, nf_flat):
    mesh = plsc.VectorSubcoreMesh(core_axis_name="c", subcore_axis_name="s")
    k = functools.partial(
        pl.kernel,
        out_type=jax.ShapeDtypeStruct((P * OUT_ROW,), jnp.float32),
        mesh=mesh,
        compiler_params=pltpu.CompilerParams(needs_layout_passes=False,
                                             use_tc_tiling_on_sc=False),
        scratch_types=[
            pltpu.VMEM((IDXC,), jnp.int32),
            pltpu.VMEM((N_FIELDS, CHUNK), jnp.int32),
            pltpu.VMEM((IDXC, EMB_DIM), jnp.float32),
            pltpu.VMEM((NF_WORDS + 16,), jnp.float32),
            pltpu.VMEM((ASM_WORDS,), jnp.float32),
            pltpu.SemaphoreType.DMA,
        ],
    )(_sc_body)
    return k(table_flat, c_flat, nf_flat)


def kernel(n_features, c_features, tables):
    table_flat = tables.reshape(N_FIELDS * VOCAB, EMB_DIM)
    c_flat = c_features.astype(jnp.int32).reshape(-1)
    nf_flat = n_features.astype(jnp.float32).reshape(-1)
    out = _concat_emb(table_flat, c_flat, nf_flat)
    return out.reshape(B, L, OUT_ROW)
